# baseline (speedup 1.0000x reference)
import functools

import jax
import jax.numpy as jnp
from jax import lax
from jax.experimental import pallas as pl
from jax.experimental.pallas import tpu as pltpu
from jax.experimental.pallas import tpu_sc as plsc

HASH = 5001
D = 64
DP = 128
NF = 6
B = 16384
DOUT = 256

NC, NS, L = 2, 16, 16
NW = NC * NS
BPW = B // NW
RCH = 64
NCH = BPW // RCH
NBUF = 4

PGRP = 16
PCH = 64


def _k1_body(rel_id, fork_num, block_num, relfilenode, rel_kind,
             p_rel_id, p_fork_num, p_block_num, p_relfilenode, p_rel_kind,
             o0, o1, o2, o3, o4,
             idx_v, buf0, buf1, buf2, buf3, sem0, sem1, sem2, sem3):
    cid = lax.axis_index("c")
    sid = lax.axis_index("s")
    wid = sid * NC + cid
    base = wid * BPW

    idx_refs = (rel_id, fork_num, block_num, relfilenode, rel_kind)
    tabs = (p_rel_id, p_fork_num, p_block_num, p_relfilenode, p_rel_kind)
    outs = (o0, o1, o2, o3, o4)
    bufs = (buf0, buf1, buf2, buf3)
    sems = (sem0, sem1, sem2, sem3)

    def stage_chunk(j, c):
        for f in range(5):
            pltpu.sync_copy(idx_refs[f].at[pl.ds(base + j * RCH, RCH)],
                            idx_v.at[f, j])
        for f in range(4):
            for i in range(RCH // L):
                v = idx_v[f, j, pl.ds(i * L, L)]
                idx_v[f, j, pl.ds(i * L, L)] = lax.rem(v, jnp.int32(HASH))
        return c

    lax.fori_loop(0, NCH, stage_chunk, 0)

    units = [(f, j) for f in range(5) for j in range(NCH)]

    def gfire(u, b):
        f, j = units[u]
        return pltpu.async_copy(tabs[f].at[idx_v.at[f, j]], bufs[b],
                                sems[b])

    def wdesc(u, b):
        f, j = units[u]
        return pltpu.make_async_copy(
            bufs[b], outs[f].at[pl.ds(base + j * RCH, RCH)], sems[b])

    pend_g = {u: gfire(u, u % NBUF) for u in range(NBUF)}
    pend_w = {}
    for u in range(len(units)):
        b = u % NBUF
        pend_g.pop(u).wait()
        pend_w[u] = wdesc(u, b)
        pend_w[u].start()
        if u + NBUF < len(units):
            pend_w.pop(u).wait()
            pend_g[u + NBUF] = gfire(u + NBUF, b)
    for u in list(pend_w):
        pend_w.pop(u).wait()


_k1 = functools.partial(
    pl.kernel,
    out_type=tuple(jax.ShapeDtypeStruct((B, DP), jnp.float32)
                   for _ in range(5)),
    mesh=plsc.VectorSubcoreMesh(core_axis_name="c", subcore_axis_name="s"),
    scratch_types=[pltpu.VMEM((5, NCH, RCH), jnp.int32)]
    + [pltpu.VMEM((RCH, DP), jnp.float32) for _ in range(NBUF)]
    + [pltpu.SemaphoreType.DMA for _ in range(NBUF)],
)(_k1_body)


def _k2_body(position, t_position, o5,
             pos_vidx, tb0, tb1, posbuf, psem0, psem1):
    cid = lax.axis_index("c")
    sid = lax.axis_index("s")
    wid = sid * NC + cid
    base = wid * BPW

    pltpu.sync_copy(position.at[pl.ds(base, BPW)], pos_vidx)

    tbs = (tb0, tb1)
    psems = (psem0, psem1)

    def fire_group(g, tb, psem):
        gv = pos_vidx[pl.ds(g * PGRP, PGRP)]
        for s in range(PGRP):
            idx = gv[s]
            tile = pl.multiple_of((idx // 8) * 8, 8)
            pltpu.async_copy(t_position.at[pl.ds(tile, 8)], tb.at[s], psem)
        return gv

    def drain_group(g, gv, tb, psem, row0):
        for s in range(PGRP):
            pltpu.make_async_copy(t_position.at[pl.ds(0, 8)], tb.at[s],
                                  psem).wait()
            sub = gv[s] % 8
            for k in range(D // L):
                posbuf[row0 + s, pl.ds(k * L, L)] = tb[s, sub, pl.ds(k * L, L)]

    def pos_chunk(jc, c):
        g0 = jc * (PCH // PGRP)
        gv_prev = fire_group(g0, tbs[0], psems[0])
        for q in range(PCH // PGRP):
            parity = q % 2
            nxt = 1 - parity
            if q + 1 < PCH // PGRP:
                gv_next = fire_group(g0 + q + 1, tbs[nxt], psems[nxt])
            drain_group(g0 + q, gv_prev, tbs[parity], psems[parity],
                        q * PGRP)
            if q + 1 < PCH // PGRP:
                gv_prev = gv_next
        pltpu.sync_copy(posbuf, o5.at[pl.ds(base + jc * PCH, PCH)])
        return c

    lax.fori_loop(0, BPW // PCH, pos_chunk, 0)


_k2 = functools.partial(
    pl.kernel,
    out_type=jax.ShapeDtypeStruct((B, D), jnp.float32),
    mesh=plsc.VectorSubcoreMesh(core_axis_name="c", subcore_axis_name="s"),
    scratch_types=[
        pltpu.VMEM((BPW,), jnp.int32),
        pltpu.VMEM((PGRP, 8, D), jnp.float32),
        pltpu.VMEM((PGRP, 8, D), jnp.float32),
        pltpu.VMEM((PCH, D), jnp.float32),
        pltpu.SemaphoreType.DMA,
        pltpu.SemaphoreType.DMA,
    ],
)(_k2_body)


BM = 1024


def _tc_gather_body(idx_sref, tpos_ref, o_ref, rows, sem):
    i = pl.program_id(0)

    def issue(k, c):
        idx = idx_sref[i * BM + k]
        pltpu.make_async_copy(tpos_ref.at[pl.ds(idx, 1), :],
                              rows.at[pl.ds(k, 1), :], sem).start()
        return c

    lax.fori_loop(0, BM, issue, 0)

    def drain(k, c):
        idx = idx_sref[i * BM + k]
        pltpu.make_async_copy(tpos_ref.at[pl.ds(idx, 1), :],
                              rows.at[pl.ds(k, 1), :], sem).wait()
        return c

    lax.fori_loop(0, BM, drain, 0)
    o_ref[...] = rows[...]


def _tc_gather(position, t_position):
    return pl.pallas_call(
        _tc_gather_body,
        grid_spec=pltpu.PrefetchScalarGridSpec(
            num_scalar_prefetch=1,
            grid=(B // BM,),
            in_specs=[pl.BlockSpec(memory_space=pl.ANY)],
            out_specs=pl.BlockSpec((BM, D), lambda i, *_: (i, 0)),
            scratch_shapes=[pltpu.VMEM((BM, D), jnp.float32),
                            pltpu.SemaphoreType.DMA],
        ),
        out_shape=jax.ShapeDtypeStruct((B, D), jnp.float32),
    )(position, t_position)


def _dense_body(x0, x1, x2, x3, x4, x5, w_ref, b_ref, o_ref):
    xs = (x0, x1, x2, x3, x4)
    acc = b_ref[...].astype(jnp.float32)
    for f in range(5):
        acc = acc + jnp.dot(xs[f][...], w_ref[f],
                            preferred_element_type=jnp.float32)
    acc = acc + jnp.dot(x5[...], w_ref[5, :D],
                        preferred_element_type=jnp.float32)
    o_ref[...] = jnp.maximum(acc, 0.0)


def _dense(xs, w3d, b2d):
    return pl.pallas_call(
        _dense_body,
        grid=(B // BM,),
        in_specs=[pl.BlockSpec((BM, DP), lambda i: (i, 0)) for _ in range(5)]
        + [
            pl.BlockSpec((BM, D), lambda i: (i, 0)),
            pl.BlockSpec((NF, DP, DOUT), lambda i: (0, 0, 0)),
            pl.BlockSpec((1, DOUT), lambda i: (0, 0)),
        ],
        out_specs=pl.BlockSpec((BM, DOUT), lambda i: (i, 0)),
        out_shape=jax.ShapeDtypeStruct((B, DOUT), jnp.float32),
    )(*xs, w3d, b2d)


def kernel(rel_id, fork_num, block_num, relfilenode, rel_kind, position,
           t_rel_id, t_fork_num, t_block_num, t_relfilenode, t_rel_kind,
           t_position, W, b):
    pad = lambda t: jnp.pad(t, ((0, 0), (0, DP - D)))
    xs = _k1(rel_id, fork_num, block_num, relfilenode, rel_kind,
             pad(t_rel_id), pad(t_fork_num), pad(t_block_num),
             pad(t_relfilenode), pad(t_rel_kind))
    x5 = _tc_gather(position, t_position)
    w3d = jnp.pad(W.reshape(NF, D, DOUT), ((0, 0), (0, DP - D), (0, 0)))
    return _dense(xs + (x5,), w3d, b.reshape(1, DOUT))


# speedup vs baseline: 1.8235x; 1.8235x over previous
import functools

import jax
import jax.numpy as jnp
from jax import lax
from jax.experimental import pallas as pl
from jax.experimental.pallas import tpu as pltpu
from jax.experimental.pallas import tpu_sc as plsc

HASH = 5001
D = 64
DP = 128
NF = 6
B = 16384
DOUT = 256

NC, NS, L = 2, 16, 16
NW = NC * NS
BPW = B // NW
RCH = 64
NCH = BPW // RCH
NBUF = 4
NSTR = 3

PGRP = 16
PCH = 64


def _k1_body(rel_id, block_num, relfilenode,
             p_rel_id, p_block_num, p_relfilenode,
             o0, o1, o2,
             idx_v, buf0, buf1, buf2, buf3, sem0, sem1, sem2, sem3):
    cid = lax.axis_index("c")
    sid = lax.axis_index("s")
    wid = sid * NC + cid
    base = wid * BPW

    idx_refs = (rel_id, block_num, relfilenode)
    tabs = (p_rel_id, p_block_num, p_relfilenode)
    outs = (o0, o1, o2)
    bufs = (buf0, buf1, buf2, buf3)
    sems = (sem0, sem1, sem2, sem3)

    def stage_chunk(j, c):
        for f in range(NSTR):
            pltpu.sync_copy(idx_refs[f].at[pl.ds(base + j * RCH, RCH)],
                            idx_v.at[f, j])
        for f in range(NSTR):
            for i in range(RCH // L):
                v = idx_v[f, j, pl.ds(i * L, L)]
                idx_v[f, j, pl.ds(i * L, L)] = lax.rem(v, jnp.int32(HASH))
        return c

    lax.fori_loop(0, NCH, stage_chunk, 0)

    units = [(f, j) for f in range(NSTR) for j in range(NCH)]
    n = len(units)

    def gfire(u):
        f, j = units[u]
        return pltpu.async_copy(tabs[f].at[idx_v.at[f, j]], bufs[u % NBUF],
                                sems[u % NBUF])

    def wdesc(u):
        f, j = units[u]
        return pltpu.make_async_copy(
            bufs[u % NBUF], outs[f].at[pl.ds(base + j * RCH, RCH)],
            sems[u % NBUF])

    pend_g = {u: gfire(u) for u in range(min(NBUF - 1, n))}
    pend_w = {}
    for u in range(n):
        pend_g.pop(u).wait()
        pend_w[u] = wdesc(u)
        pend_w[u].start()
        v = u + NBUF - 1
        if v < n:
            if u > 0:
                pend_w.pop(u - 1).wait()
            pend_g[v] = gfire(v)
    for u in list(pend_w):
        pend_w.pop(u).wait()


_k1 = functools.partial(
    pl.kernel,
    out_type=tuple(jax.ShapeDtypeStruct((B, DP), jnp.float32)
                   for _ in range(NSTR)),
    mesh=plsc.VectorSubcoreMesh(core_axis_name="c", subcore_axis_name="s"),
    scratch_types=[pltpu.VMEM((NSTR, NCH, RCH), jnp.int32)]
    + [pltpu.VMEM((RCH, DP), jnp.float32) for _ in range(NBUF)]
    + [pltpu.SemaphoreType.DMA for _ in range(NBUF)],
)(_k1_body)


def _k2_body(position, t_position, o5,
             pos_vidx, tb0, tb1, posbuf, psem0, psem1):
    cid = lax.axis_index("c")
    sid = lax.axis_index("s")
    wid = sid * NC + cid
    base = wid * BPW

    pltpu.sync_copy(position.at[pl.ds(base, BPW)], pos_vidx)

    tbs = (tb0, tb1)
    psems = (psem0, psem1)

    def fire_group(g, tb, psem):
        gv = pos_vidx[pl.ds(g * PGRP, PGRP)]
        for s in range(PGRP):
            idx = gv[s]
            tile = pl.multiple_of((idx // 8) * 8, 8)
            pltpu.async_copy(t_position.at[pl.ds(tile, 8)], tb.at[s], psem)
        return gv

    def drain_group(g, gv, tb, psem, row0):
        for s in range(PGRP):
            pltpu.make_async_copy(t_position.at[pl.ds(0, 8)], tb.at[s],
                                  psem).wait()
            sub = gv[s] % 8
            for k in range(D // L):
                posbuf[row0 + s, pl.ds(k * L, L)] = tb[s, sub, pl.ds(k * L, L)]

    def pos_chunk(jc, c):
        g0 = jc * (PCH // PGRP)
        gv_prev = fire_group(g0, tbs[0], psems[0])
        for q in range(PCH // PGRP):
            parity = q % 2
            nxt = 1 - parity
            if q + 1 < PCH // PGRP:
                gv_next = fire_group(g0 + q + 1, tbs[nxt], psems[nxt])
            drain_group(g0 + q, gv_prev, tbs[parity], psems[parity],
                        q * PGRP)
            if q + 1 < PCH // PGRP:
                gv_prev = gv_next
        pltpu.sync_copy(posbuf, o5.at[pl.ds(base + jc * PCH, PCH)])
        return c

    lax.fori_loop(0, BPW // PCH, pos_chunk, 0)


_k2 = functools.partial(
    pl.kernel,
    out_type=jax.ShapeDtypeStruct((B, D), jnp.float32),
    mesh=plsc.VectorSubcoreMesh(core_axis_name="c", subcore_axis_name="s"),
    scratch_types=[
        pltpu.VMEM((BPW,), jnp.int32),
        pltpu.VMEM((PGRP, 8, D), jnp.float32),
        pltpu.VMEM((PGRP, 8, D), jnp.float32),
        pltpu.VMEM((PCH, D), jnp.float32),
        pltpu.SemaphoreType.DMA,
        pltpu.SemaphoreType.DMA,
    ],
)(_k2_body)


BM = 1024

NFORK = 8
NKIND = 16


def _dense_body(fork_i, kind_i, x0, x1, x2, x5, f8, k16, w_ref, b_ref, o_ref):
    acc = b_ref[...].astype(jnp.float32)
    for f, x in ((0, x0), (2, x1), (3, x2)):
        acc = acc + jnp.dot(x[...], w_ref[f],
                            preferred_element_type=jnp.float32)
    acc = acc + jnp.dot(x5[...], w_ref[5, :D],
                        preferred_element_type=jnp.float32)
    p_fork = jnp.dot(f8[...], w_ref[1, :D], preferred_element_type=jnp.float32)
    p_kind = jnp.dot(k16[...], w_ref[4, :D], preferred_element_type=jnp.float32)
    oh_f = (fork_i[...] == lax.broadcasted_iota(jnp.int32, (BM, NFORK), 1))
    oh_k = (kind_i[...] == lax.broadcasted_iota(jnp.int32, (BM, NKIND), 1))
    acc = acc + jnp.dot(oh_f.astype(jnp.float32), p_fork,
                        preferred_element_type=jnp.float32)
    acc = acc + jnp.dot(oh_k.astype(jnp.float32), p_kind,
                        preferred_element_type=jnp.float32)
    o_ref[...] = jnp.maximum(acc, 0.0)


def _dense(fork2d, kind2d, xs, x5, f8, k16, w3d, b2d):
    return pl.pallas_call(
        _dense_body,
        grid=(B // BM,),
        in_specs=[
            pl.BlockSpec((BM, 1), lambda i: (i, 0)),
            pl.BlockSpec((BM, 1), lambda i: (i, 0)),
        ]
        + [pl.BlockSpec((BM, DP), lambda i: (i, 0)) for _ in range(NSTR)]
        + [
            pl.BlockSpec((BM, D), lambda i: (i, 0)),
            pl.BlockSpec((NFORK, D), lambda i: (0, 0)),
            pl.BlockSpec((NKIND, D), lambda i: (0, 0)),
            pl.BlockSpec((NF, DP, DOUT), lambda i: (0, 0, 0)),
            pl.BlockSpec((1, DOUT), lambda i: (0, 0)),
        ],
        out_specs=pl.BlockSpec((BM, DOUT), lambda i: (i, 0)),
        out_shape=jax.ShapeDtypeStruct((B, DOUT), jnp.float32),
    )(fork2d, kind2d, *xs, x5, f8, k16, w3d, b2d)


def kernel(rel_id, fork_num, block_num, relfilenode, rel_kind, position,
           t_rel_id, t_fork_num, t_block_num, t_relfilenode, t_rel_kind,
           t_position, W, b):
    pad = lambda t: jnp.pad(t, ((0, 0), (0, DP - D)))
    xs = _k1(rel_id, block_num, relfilenode,
             pad(t_rel_id), pad(t_block_num), pad(t_relfilenode))
    x5 = _k2(position, t_position)
    f8 = lax.slice(t_fork_num, (0, 0), (NFORK, D))
    k16 = jnp.pad(t_rel_kind, ((0, NKIND - 10), (0, 0)))
    w3d = jnp.pad(W.reshape(NF, D, DOUT), ((0, 0), (0, DP - D), (0, 0)))
    return _dense(fork_num.reshape(B, 1), rel_kind.reshape(B, 1),
                  xs, x5, f8, k16, w3d, b.reshape(1, DOUT))


# speedup vs baseline: 1.8294x; 1.0032x over previous
import functools

import jax
import jax.numpy as jnp
from jax import lax
from jax.experimental import pallas as pl
from jax.experimental.pallas import tpu as pltpu
from jax.experimental.pallas import tpu_sc as plsc

HASH = 5001
D = 64
DP = 128
NF = 6
B = 16384
DOUT = 256

NC, NS, L = 2, 16, 16
NW = NC * NS
BPW = B // NW
RCH = 64
NCH = BPW // RCH
NBUF = 4
NSTR = 3

PGRP = 32
PCH = 64


def _k1_body(rel_id, block_num, relfilenode,
             p_rel_id, p_block_num, p_relfilenode,
             o0, o1, o2,
             idx_v, buf0, buf1, buf2, buf3, sem0, sem1, sem2, sem3):
    cid = lax.axis_index("c")
    sid = lax.axis_index("s")
    wid = sid * NC + cid
    base = wid * BPW

    idx_refs = (rel_id, block_num, relfilenode)
    tabs = (p_rel_id, p_block_num, p_relfilenode)
    outs = (o0, o1, o2)
    bufs = (buf0, buf1, buf2, buf3)
    sems = (sem0, sem1, sem2, sem3)

    def stage_chunk(j, c):
        for f in range(NSTR):
            pltpu.sync_copy(idx_refs[f].at[pl.ds(base + j * RCH, RCH)],
                            idx_v.at[f, j])
        for f in range(NSTR):
            for i in range(RCH // L):
                v = idx_v[f, j, pl.ds(i * L, L)]
                idx_v[f, j, pl.ds(i * L, L)] = lax.rem(v, jnp.int32(HASH))
        return c

    lax.fori_loop(0, NCH, stage_chunk, 0)

    units = [(f, j) for f in range(NSTR) for j in range(NCH)]
    n = len(units)

    def gfire(u):
        f, j = units[u]
        return pltpu.async_copy(tabs[f].at[idx_v.at[f, j]], bufs[u % NBUF],
                                sems[u % NBUF])

    def wdesc(u):
        f, j = units[u]
        return pltpu.make_async_copy(
            bufs[u % NBUF], outs[f].at[pl.ds(base + j * RCH, RCH)],
            sems[u % NBUF])

    pend_g = {u: gfire(u) for u in range(min(NBUF - 1, n))}
    pend_w = {}
    for u in range(n):
        pend_g.pop(u).wait()
        pend_w[u] = wdesc(u)
        pend_w[u].start()
        v = u + NBUF - 1
        if v < n:
            if u > 0:
                pend_w.pop(u - 1).wait()
            pend_g[v] = gfire(v)
    for u in list(pend_w):
        pend_w.pop(u).wait()


_k1 = functools.partial(
    pl.kernel,
    out_type=tuple(jax.ShapeDtypeStruct((B, DP), jnp.float32)
                   for _ in range(NSTR)),
    mesh=plsc.VectorSubcoreMesh(core_axis_name="c", subcore_axis_name="s"),
    scratch_types=[pltpu.VMEM((NSTR, NCH, RCH), jnp.int32)]
    + [pltpu.VMEM((RCH, DP), jnp.float32) for _ in range(NBUF)]
    + [pltpu.SemaphoreType.DMA for _ in range(NBUF)],
)(_k1_body)


def _k2_body(position, t_position, o5,
             pos_vidx, tb0, tb1, posbuf, psem0, psem1):
    cid = lax.axis_index("c")
    sid = lax.axis_index("s")
    wid = sid * NC + cid
    base = wid * BPW

    pltpu.sync_copy(position.at[pl.ds(base, BPW)], pos_vidx)

    tbs = (tb0, tb1)
    psems = (psem0, psem1)

    def fire_group(g, tb, psem):
        gv = pos_vidx[pl.ds(g * PGRP, PGRP)]
        for s in range(PGRP):
            idx = gv[s]
            tile = pl.multiple_of((idx // 8) * 8, 8)
            pltpu.async_copy(t_position.at[pl.ds(tile, 8)], tb.at[s], psem)
        return gv

    def drain_group(g, gv, tb, psem, row0):
        for s in range(PGRP):
            pltpu.make_async_copy(t_position.at[pl.ds(0, 8)], tb.at[s],
                                  psem).wait()
            sub = gv[s] % 8
            for k in range(D // L):
                posbuf[row0 + s, pl.ds(k * L, L)] = tb[s, sub, pl.ds(k * L, L)]

    def pos_chunk(jc, c):
        g0 = jc * (PCH // PGRP)
        gv_prev = fire_group(g0, tbs[0], psems[0])
        for q in range(PCH // PGRP):
            parity = q % 2
            nxt = 1 - parity
            if q + 1 < PCH // PGRP:
                gv_next = fire_group(g0 + q + 1, tbs[nxt], psems[nxt])
            drain_group(g0 + q, gv_prev, tbs[parity], psems[parity],
                        q * PGRP)
            if q + 1 < PCH // PGRP:
                gv_prev = gv_next
        pltpu.sync_copy(posbuf, o5.at[pl.ds(base + jc * PCH, PCH)])
        return c

    lax.fori_loop(0, BPW // PCH, pos_chunk, 0)


_k2 = functools.partial(
    pl.kernel,
    out_type=jax.ShapeDtypeStruct((B, D), jnp.float32),
    mesh=plsc.VectorSubcoreMesh(core_axis_name="c", subcore_axis_name="s"),
    scratch_types=[
        pltpu.VMEM((BPW,), jnp.int32),
        pltpu.VMEM((PGRP, 8, D), jnp.float32),
        pltpu.VMEM((PGRP, 8, D), jnp.float32),
        pltpu.VMEM((PCH, D), jnp.float32),
        pltpu.SemaphoreType.DMA,
        pltpu.SemaphoreType.DMA,
    ],
)(_k2_body)


BM = 2048

NFORK = 8
NKIND = 16


def _dense_body(fork_i, kind_i, x0, x1, x2, x5, f8, k16, w_ref, b_ref, o_ref):
    acc = b_ref[...].astype(jnp.float32)
    for f, x in ((0, x0), (2, x1), (3, x2)):
        acc = acc + jnp.dot(x[...], w_ref[f],
                            preferred_element_type=jnp.float32)
    acc = acc + jnp.dot(x5[...], w_ref[5, :D],
                        preferred_element_type=jnp.float32)
    p_fork = jnp.dot(f8[...], w_ref[1, :D], preferred_element_type=jnp.float32)
    p_kind = jnp.dot(k16[...], w_ref[4, :D], preferred_element_type=jnp.float32)
    oh_f = (fork_i[...] == lax.broadcasted_iota(jnp.int32, (BM, NFORK), 1))
    oh_k = (kind_i[...] == lax.broadcasted_iota(jnp.int32, (BM, NKIND), 1))
    acc = acc + jnp.dot(oh_f.astype(jnp.float32), p_fork,
                        preferred_element_type=jnp.float32)
    acc = acc + jnp.dot(oh_k.astype(jnp.float32), p_kind,
                        preferred_element_type=jnp.float32)
    o_ref[...] = jnp.maximum(acc, 0.0)


def _dense(fork2d, kind2d, xs, x5, f8, k16, w3d, b2d):
    return pl.pallas_call(
        _dense_body,
        grid=(B // BM,),
        in_specs=[
            pl.BlockSpec((BM, 1), lambda i: (i, 0)),
            pl.BlockSpec((BM, 1), lambda i: (i, 0)),
        ]
        + [pl.BlockSpec((BM, DP), lambda i: (i, 0)) for _ in range(NSTR)]
        + [
            pl.BlockSpec((BM, D), lambda i: (i, 0)),
            pl.BlockSpec((NFORK, D), lambda i: (0, 0)),
            pl.BlockSpec((NKIND, D), lambda i: (0, 0)),
            pl.BlockSpec((NF, DP, DOUT), lambda i: (0, 0, 0)),
            pl.BlockSpec((1, DOUT), lambda i: (0, 0)),
        ],
        out_specs=pl.BlockSpec((BM, DOUT), lambda i: (i, 0)),
        out_shape=jax.ShapeDtypeStruct((B, DOUT), jnp.float32),
    )(fork2d, kind2d, *xs, x5, f8, k16, w3d, b2d)


def kernel(rel_id, fork_num, block_num, relfilenode, rel_kind, position,
           t_rel_id, t_fork_num, t_block_num, t_relfilenode, t_rel_kind,
           t_position, W, b):
    pad = lambda t: jnp.pad(t, ((0, 0), (0, DP - D)))
    xs = _k1(rel_id, block_num, relfilenode,
             pad(t_rel_id), pad(t_block_num), pad(t_relfilenode))
    x5 = _k2(position, t_position)
    f8 = lax.slice(t_fork_num, (0, 0), (NFORK, D))
    k16 = jnp.pad(t_rel_kind, ((0, NKIND - 10), (0, 0)))
    w3d = jnp.pad(W.reshape(NF, D, DOUT), ((0, 0), (0, DP - D), (0, 0)))
    return _dense(fork_num.reshape(B, 1), rel_kind.reshape(B, 1),
                  xs, x5, f8, k16, w3d, b.reshape(1, DOUT))


# speedup vs baseline: 1.8497x; 1.0111x over previous
import functools

import jax
import jax.numpy as jnp
from jax import lax
from jax.experimental import pallas as pl
from jax.experimental.pallas import tpu as pltpu
from jax.experimental.pallas import tpu_sc as plsc

HASH = 5001
D = 64
DP = 128
NF = 6
B = 16384
DOUT = 256

NC, NS, L = 2, 16, 16
NW = NC * NS
BPW = B // NW
RCH = 64
NCH = BPW // RCH
NBUF = 4
NSTR = 3

PGRP = 32
PCH = 64


def _k1_body(rel_id, block_num, relfilenode,
             p_rel_id, p_block_num, p_relfilenode,
             o0, o1, o2,
             idx_v, buf0, buf1, buf2, buf3, sem0, sem1, sem2, sem3):
    cid = lax.axis_index("c")
    sid = lax.axis_index("s")
    wid = sid * NC + cid
    base = wid * BPW

    idx_refs = (rel_id, block_num, relfilenode)
    tabs = (p_rel_id, p_block_num, p_relfilenode)
    outs = (o0, o1, o2)
    bufs = (buf0, buf1, buf2, buf3)
    sems = (sem0, sem1, sem2, sem3)

    def stage_chunk(j, c):
        for f in range(NSTR):
            pltpu.sync_copy(idx_refs[f].at[pl.ds(base + j * RCH, RCH)],
                            idx_v.at[f, j])
        for f in range(NSTR):
            for i in range(RCH // L):
                v = idx_v[f, j, pl.ds(i * L, L)]
                idx_v[f, j, pl.ds(i * L, L)] = lax.rem(v, jnp.int32(HASH))
        return c

    lax.fori_loop(0, NCH, stage_chunk, 0)

    units = [(f, j) for f in range(NSTR) for j in range(NCH)]
    n = len(units)

    def gfire(u):
        f, j = units[u]
        return pltpu.async_copy(tabs[f].at[idx_v.at[f, j]], bufs[u % NBUF],
                                sems[u % NBUF])

    def wdesc(u):
        f, j = units[u]
        return pltpu.make_async_copy(
            bufs[u % NBUF], outs[f].at[pl.ds(base + j * RCH, RCH)],
            sems[u % NBUF])

    pend_g = {u: gfire(u) for u in range(min(NBUF - 1, n))}
    pend_w = {}
    for u in range(n):
        pend_g.pop(u).wait()
        pend_w[u] = wdesc(u)
        pend_w[u].start()
        v = u + NBUF - 1
        if v < n:
            if u > 0:
                pend_w.pop(u - 1).wait()
            pend_g[v] = gfire(v)
    for u in list(pend_w):
        pend_w.pop(u).wait()


_k1 = functools.partial(
    pl.kernel,
    out_type=tuple(jax.ShapeDtypeStruct((B, DP), jnp.float32)
                   for _ in range(NSTR)),
    mesh=plsc.VectorSubcoreMesh(core_axis_name="c", subcore_axis_name="s"),
    scratch_types=[pltpu.VMEM((NSTR, NCH, RCH), jnp.int32)]
    + [pltpu.VMEM((RCH, DP), jnp.float32) for _ in range(NBUF)]
    + [pltpu.SemaphoreType.DMA for _ in range(NBUF)],
)(_k1_body)


def _k2_body(position, t_position, o5,
             pos_vidx, tb0, tb1, posbuf, psem0, psem1):
    cid = lax.axis_index("c")
    sid = lax.axis_index("s")
    wid = sid * NC + cid
    base = wid * BPW

    pltpu.sync_copy(position.at[pl.ds(base, BPW)], pos_vidx)

    tbs = (tb0, tb1)
    psems = (psem0, psem1)

    def fire_group(g, tb, psem):
        gv = pos_vidx[pl.ds(g * PGRP, PGRP)]
        for s in range(PGRP):
            idx = gv[s]
            tile = pl.multiple_of((idx // 8) * 8, 8)
            pltpu.async_copy(t_position.at[pl.ds(tile, 8)], tb.at[s], psem)
        return gv

    def drain_group(g, gv, tb, psem, row0):
        for s in range(PGRP):
            pltpu.make_async_copy(t_position.at[pl.ds(0, 8)], tb.at[s],
                                  psem).wait()
            sub = gv[s] % 8
            for k in range(D // L):
                posbuf[row0 + s, pl.ds(k * L, L)] = tb[s, sub, pl.ds(k * L, L)]

    def pos_chunk(jc, c):
        g0 = jc * (PCH // PGRP)
        gv_prev = fire_group(g0, tbs[0], psems[0])
        for q in range(PCH // PGRP):
            parity = q % 2
            nxt = 1 - parity
            if q + 1 < PCH // PGRP:
                gv_next = fire_group(g0 + q + 1, tbs[nxt], psems[nxt])
            drain_group(g0 + q, gv_prev, tbs[parity], psems[parity],
                        q * PGRP)
            if q + 1 < PCH // PGRP:
                gv_prev = gv_next
        pltpu.sync_copy(posbuf, o5.at[pl.ds(base + jc * PCH, PCH)])
        return c

    lax.fori_loop(0, BPW // PCH, pos_chunk, 0)


_k2 = functools.partial(
    pl.kernel,
    out_type=jax.ShapeDtypeStruct((B, D), jnp.float32),
    mesh=plsc.VectorSubcoreMesh(core_axis_name="c", subcore_axis_name="s"),
    scratch_types=[
        pltpu.VMEM((BPW,), jnp.int32),
        pltpu.VMEM((PGRP, 8, D), jnp.float32),
        pltpu.VMEM((PGRP, 8, D), jnp.float32),
        pltpu.VMEM((PCH, D), jnp.float32),
        pltpu.SemaphoreType.DMA,
        pltpu.SemaphoreType.DMA,
    ],
)(_k2_body)


BM = 2048

NFORK = 8
NKIND = 16


def _dense_a_body(fork_i, kind_i, x0, x1, x2, f8, k16, w_ref, b_ref,
                  o_ref):
    acc = b_ref[...].astype(jnp.float32)
    for f, x in ((0, x0), (2, x1), (3, x2)):
        acc = acc + jnp.dot(x[...], w_ref[f],
                            preferred_element_type=jnp.float32)
    p_fork = jnp.dot(f8[...], w_ref[1, :D], preferred_element_type=jnp.float32)
    p_kind = jnp.dot(k16[...], w_ref[4, :D], preferred_element_type=jnp.float32)
    oh_f = (fork_i[...] == lax.broadcasted_iota(jnp.int32, (BM, NFORK), 1))
    oh_k = (kind_i[...] == lax.broadcasted_iota(jnp.int32, (BM, NKIND), 1))
    acc = acc + jnp.dot(oh_f.astype(jnp.float32), p_fork,
                        preferred_element_type=jnp.float32)
    acc = acc + jnp.dot(oh_k.astype(jnp.float32), p_kind,
                        preferred_element_type=jnp.float32)
    o_ref[...] = acc


def _dense_a(fork2d, kind2d, xs, f8, k16, w3d, b2d):
    return pl.pallas_call(
        _dense_a_body,
        grid=(B // BM,),
        in_specs=[
            pl.BlockSpec((BM, 1), lambda i: (i, 0)),
            pl.BlockSpec((BM, 1), lambda i: (i, 0)),
        ]
        + [pl.BlockSpec((BM, DP), lambda i: (i, 0)) for _ in range(NSTR)]
        + [
            pl.BlockSpec((NFORK, D), lambda i: (0, 0)),
            pl.BlockSpec((NKIND, D), lambda i: (0, 0)),
            pl.BlockSpec((NF, DP, DOUT), lambda i: (0, 0, 0)),
            pl.BlockSpec((1, DOUT), lambda i: (0, 0)),
        ],
        out_specs=pl.BlockSpec((BM, DOUT), lambda i: (i, 0)),
        out_shape=jax.ShapeDtypeStruct((B, DOUT), jnp.float32),
    )(fork2d, kind2d, *xs, f8, k16, w3d, b2d)


def _dense_b_body(acc_ref, x5, w5_ref, o_ref):
    acc = acc_ref[...] + jnp.dot(x5[...], w5_ref[...],
                                 preferred_element_type=jnp.float32)
    o_ref[...] = jnp.maximum(acc, 0.0)


def _dense_b(acc, x5, w5):
    return pl.pallas_call(
        _dense_b_body,
        grid=(B // BM,),
        in_specs=[
            pl.BlockSpec((BM, DOUT), lambda i: (i, 0)),
            pl.BlockSpec((BM, D), lambda i: (i, 0)),
            pl.BlockSpec((D, DOUT), lambda i: (0, 0)),
        ],
        out_specs=pl.BlockSpec((BM, DOUT), lambda i: (i, 0)),
        out_shape=jax.ShapeDtypeStruct((B, DOUT), jnp.float32),
    )(acc, x5, w5)


def kernel(rel_id, fork_num, block_num, relfilenode, rel_kind, position,
           t_rel_id, t_fork_num, t_block_num, t_relfilenode, t_rel_kind,
           t_position, W, b):
    pad = lambda t: jnp.pad(t, ((0, 0), (0, DP - D)))
    xs = _k1(rel_id, block_num, relfilenode,
             pad(t_rel_id), pad(t_block_num), pad(t_relfilenode))
    x5 = _k2(position, t_position)
    f8 = lax.slice(t_fork_num, (0, 0), (NFORK, D))
    k16 = jnp.pad(t_rel_kind, ((0, NKIND - 10), (0, 0)))
    w3d = jnp.pad(W.reshape(NF, D, DOUT), ((0, 0), (0, DP - D), (0, 0)))
    acc = _dense_a(fork_num.reshape(B, 1), rel_kind.reshape(B, 1),
                   xs, f8, k16, w3d, b.reshape(1, DOUT))
    return _dense_b(acc, x5, W.reshape(NF, D, DOUT)[5])


# speedup vs baseline: 1.8730x; 1.0126x over previous
import functools

import jax
import jax.numpy as jnp
from jax import lax
from jax.experimental import pallas as pl
from jax.experimental.pallas import tpu as pltpu
from jax.experimental.pallas import tpu_sc as plsc

HASH = 5001
D = 64
DP = 128
NF = 6
B = 16384
DOUT = 256

NC, NS, L = 2, 16, 16
NW = NC * NS
BPW = B // NW
RCH = 64
NCH = BPW // RCH
NBUF = 4
NSTR = 3

PGRP = 32
PCH = 64


def _k1_body(rel_id, block_num, relfilenode,
             p_rel_id, p_block_num, p_relfilenode,
             o0, o1, o2,
             idx_v, buf0, buf1, buf2, buf3, sem0, sem1, sem2, sem3):
    cid = lax.axis_index("c")
    sid = lax.axis_index("s")
    wid = sid * NC + cid
    base = wid * BPW

    idx_refs = (rel_id, block_num, relfilenode)
    tabs = (p_rel_id, p_block_num, p_relfilenode)
    outs = (o0, o1, o2)
    bufs = (buf0, buf1, buf2, buf3)
    sems = (sem0, sem1, sem2, sem3)

    def stage_chunk(j, c):
        for f in range(NSTR):
            pltpu.sync_copy(idx_refs[f].at[pl.ds(base + j * RCH, RCH)],
                            idx_v.at[f, j])
        for f in range(NSTR):
            for i in range(RCH // L):
                v = idx_v[f, j, pl.ds(i * L, L)]
                idx_v[f, j, pl.ds(i * L, L)] = lax.rem(v, jnp.int32(HASH))
        return c

    lax.fori_loop(0, NCH, stage_chunk, 0)

    units = [(f, j) for f in range(NSTR) for j in range(NCH)]
    n = len(units)

    def gfire(u):
        f, j = units[u]
        return pltpu.async_copy(tabs[f].at[idx_v.at[f, j]], bufs[u % NBUF],
                                sems[u % NBUF])

    def wdesc(u):
        f, j = units[u]
        return pltpu.make_async_copy(
            bufs[u % NBUF], outs[f].at[pl.ds(base + j * RCH, RCH)],
            sems[u % NBUF])

    pend_g = {u: gfire(u) for u in range(min(NBUF - 1, n))}
    pend_w = {}
    for u in range(n):
        pend_g.pop(u).wait()
        pend_w[u] = wdesc(u)
        pend_w[u].start()
        v = u + NBUF - 1
        if v < n:
            if u > 0:
                pend_w.pop(u - 1).wait()
            pend_g[v] = gfire(v)
    for u in list(pend_w):
        pend_w.pop(u).wait()


_k1 = functools.partial(
    pl.kernel,
    out_type=tuple(jax.ShapeDtypeStruct((B, DP), jnp.float32)
                   for _ in range(NSTR)),
    mesh=plsc.VectorSubcoreMesh(core_axis_name="c", subcore_axis_name="s"),
    scratch_types=[pltpu.VMEM((NSTR, NCH, RCH), jnp.int32)]
    + [pltpu.VMEM((RCH, DP), jnp.float32) for _ in range(NBUF)]
    + [pltpu.SemaphoreType.DMA for _ in range(NBUF)],
)(_k1_body)


def _k2_body(position, t_position, o5,
             pos_vidx, tb0, tb1, posbuf, psem0, psem1):
    cid = lax.axis_index("c")
    sid = lax.axis_index("s")
    wid = sid * NC + cid
    base = wid * BPW

    pltpu.sync_copy(position.at[pl.ds(base, BPW)], pos_vidx)

    tbs = (tb0, tb1)
    psems = (psem0, psem1)

    def fire_group(g, tb, psem):
        gv = pos_vidx[pl.ds(g * PGRP, PGRP)]
        for s in range(PGRP):
            idx = gv[s]
            tile = pl.multiple_of((idx // 8) * 8, 8)
            pltpu.async_copy(t_position.at[pl.ds(tile, 8)], tb.at[s], psem)
        return gv

    def drain_group(g, gv, tb, psem, row0):
        for s in range(PGRP):
            pltpu.make_async_copy(t_position.at[pl.ds(0, 8)], tb.at[s],
                                  psem).wait()
            sub = gv[s] % 8
            for k in range(D // L):
                posbuf[row0 + s, pl.ds(k * L, L)] = tb[s, sub, pl.ds(k * L, L)]

    def pos_chunk(jc, c):
        g0 = jc * (PCH // PGRP)
        gv_prev = fire_group(g0, tbs[0], psems[0])
        for q in range(PCH // PGRP):
            parity = q % 2
            nxt = 1 - parity
            if q + 1 < PCH // PGRP:
                gv_next = fire_group(g0 + q + 1, tbs[nxt], psems[nxt])
            drain_group(g0 + q, gv_prev, tbs[parity], psems[parity],
                        q * PGRP)
            if q + 1 < PCH // PGRP:
                gv_prev = gv_next
        pltpu.sync_copy(posbuf, o5.at[pl.ds(base + jc * PCH, PCH)])
        return c

    lax.fori_loop(0, BPW // PCH, pos_chunk, 0)


_k2 = functools.partial(
    pl.kernel,
    out_type=jax.ShapeDtypeStruct((B, D), jnp.float32),
    mesh=plsc.VectorSubcoreMesh(core_axis_name="c", subcore_axis_name="s"),
    scratch_types=[
        pltpu.VMEM((BPW,), jnp.int32),
        pltpu.VMEM((PGRP, 8, D), jnp.float32),
        pltpu.VMEM((PGRP, 8, D), jnp.float32),
        pltpu.VMEM((PCH, D), jnp.float32),
        pltpu.SemaphoreType.DMA,
        pltpu.SemaphoreType.DMA,
    ],
)(_k2_body)


BM = 2048

NFORK = 8
NKIND = 16


def _dense_a_body(fork_i, kind_i, x0, x1, x2, f8, k16, w_ref, b_ref,
                  o_ref):
    acc = b_ref[...].astype(jnp.float32)
    for f, x in ((0, x0), (2, x1), (3, x2)):
        acc = acc + jnp.dot(x[...], w_ref[f],
                            preferred_element_type=jnp.float32)
    p_fork = jnp.dot(f8[...], w_ref[1, :D], preferred_element_type=jnp.float32)
    p_kind = jnp.dot(k16[...], w_ref[4, :D], preferred_element_type=jnp.float32)
    oh_f = (fork_i[...][:, None]
            == lax.broadcasted_iota(jnp.int32, (BM, NFORK), 1))
    oh_k = (kind_i[...][:, None]
            == lax.broadcasted_iota(jnp.int32, (BM, NKIND), 1))
    acc = acc + jnp.dot(oh_f.astype(jnp.float32), p_fork,
                        preferred_element_type=jnp.float32)
    acc = acc + jnp.dot(oh_k.astype(jnp.float32), p_kind,
                        preferred_element_type=jnp.float32)
    o_ref[...] = acc


def _dense_a(fork2d, kind2d, xs, f8, k16, w3d, b2d):
    return pl.pallas_call(
        _dense_a_body,
        grid=(B // BM,),
        in_specs=[
            pl.BlockSpec((BM,), lambda i: (i,)),
            pl.BlockSpec((BM,), lambda i: (i,)),
        ]
        + [pl.BlockSpec((BM, DP), lambda i: (i, 0)) for _ in range(NSTR)]
        + [
            pl.BlockSpec((NFORK, D), lambda i: (0, 0)),
            pl.BlockSpec((NKIND, D), lambda i: (0, 0)),
            pl.BlockSpec((NF, DP, DOUT), lambda i: (0, 0, 0)),
            pl.BlockSpec((1, DOUT), lambda i: (0, 0)),
        ],
        out_specs=pl.BlockSpec((BM, DOUT), lambda i: (i, 0)),
        out_shape=jax.ShapeDtypeStruct((B, DOUT), jnp.float32),
    )(fork2d, kind2d, *xs, f8, k16, w3d, b2d)


def _dense_b_body(acc_ref, x5, w5_ref, o_ref):
    acc = acc_ref[...] + jnp.dot(x5[...], w5_ref[...],
                                 preferred_element_type=jnp.float32)
    o_ref[...] = jnp.maximum(acc, 0.0)


def _dense_b(acc, x5, w5):
    return pl.pallas_call(
        _dense_b_body,
        grid=(B // BM,),
        in_specs=[
            pl.BlockSpec((BM, DOUT), lambda i: (i, 0)),
            pl.BlockSpec((BM, D), lambda i: (i, 0)),
            pl.BlockSpec((D, DOUT), lambda i: (0, 0)),
        ],
        out_specs=pl.BlockSpec((BM, DOUT), lambda i: (i, 0)),
        out_shape=jax.ShapeDtypeStruct((B, DOUT), jnp.float32),
    )(acc, x5, w5)


def kernel(rel_id, fork_num, block_num, relfilenode, rel_kind, position,
           t_rel_id, t_fork_num, t_block_num, t_relfilenode, t_rel_kind,
           t_position, W, b):
    pad = lambda t: jnp.pad(t, ((0, 0), (0, DP - D)))
    xs = _k1(rel_id, block_num, relfilenode,
             pad(t_rel_id), pad(t_block_num), pad(t_relfilenode))
    x5 = _k2(position, t_position)
    f8 = lax.slice(t_fork_num, (0, 0), (NFORK, D))
    k16 = jnp.pad(t_rel_kind, ((0, NKIND - 10), (0, 0)))
    w3d = jnp.pad(W.reshape(NF, D, DOUT), ((0, 0), (0, DP - D), (0, 0)))
    acc = _dense_a(fork_num, rel_kind, xs, f8, k16, w3d,
                   b.reshape(1, DOUT))
    return _dense_b(acc, x5, W.reshape(NF, D, DOUT)[5])
